# baseline (speedup 1.0000x reference)
import numpy as np
import jax
import jax.numpy as jnp
from jax import lax
from jax.experimental import pallas as pl
from jax.experimental.pallas import tpu as pltpu
from jax.experimental.pallas import tpu_sc as plsc

N = 10000
E = 320000
DF = 128
NGR = 16
NC = 2
NS = 16
NW = NC * NS
EPW = E // NW
CHUNK = 125
NCH = EPW // CHUNK
NPAD = 10240
RPS = NPAD // NS
DW = 16
BN = 1000
NB = N // BN


def _sc_mesh():
    return plsc.VectorSubcoreMesh(
        core_axis_name="c", subcore_axis_name="s", num_cores=NC, num_subcores=NS)


def _agg_body(feat, src3, dst3, znd, out0, out1, srcv, dstv, buf, accsh, sem):
    cid = lax.axis_index("c")
    sid = lax.axis_index("s")
    wid = cid * NS + sid

    pltpu.sync_copy(src3.at[wid], srcv)
    pltpu.sync_copy(dst3.at[wid], dstv)

    r0 = sid * RPS
    pltpu.sync_copy(znd.at[pl.ds(r0, RPS)], accsh.at[pl.ds(r0, RPS)])
    plsc.subcore_barrier()

    def step(j, carry):
        pltpu.async_copy(feat.at[srcv.at[j]], buf, sem).wait()
        pltpu.sync_copy(buf, accsh.at[dstv.at[j]], add=True)
        return carry

    lax.fori_loop(0, NCH, step, 0)

    plsc.subcore_barrier()
    _writeback(cid, sid, accsh, out0, out1)


def _writeback(cid, sid, sh, out0, out1):
    r0 = sid * RPS
    nrows = N - 15 * RPS

    @pl.when(cid == 0)
    def _c0():
        @pl.when(sid < NS - 1)
        def _full():
            pltpu.sync_copy(sh.at[pl.ds(r0, RPS)], out0.at[pl.ds(r0, RPS)])

        @pl.when(sid == NS - 1)
        def _last():
            pltpu.sync_copy(sh.at[pl.ds(r0, nrows)], out0.at[pl.ds(r0, nrows)])

    @pl.when(cid == 1)
    def _c1():
        @pl.when(sid < NS - 1)
        def _full():
            pltpu.sync_copy(sh.at[pl.ds(r0, RPS)], out1.at[pl.ds(r0, RPS)])

        @pl.when(sid == NS - 1)
        def _last():
            pltpu.sync_copy(sh.at[pl.ds(r0, nrows)], out1.at[pl.ds(r0, nrows)])


def _deg_body(dst3, zdg, onesc, dg0, dg1, dstv, onev, degsh):
    cid = lax.axis_index("c")
    sid = lax.axis_index("s")
    wid = cid * NS + sid

    pltpu.sync_copy(dst3.at[wid], dstv)
    r0 = sid * RPS
    pltpu.sync_copy(zdg.at[pl.ds(r0, RPS)], degsh.at[pl.ds(r0, RPS)])
    pltpu.sync_copy(onesc, onev)
    plsc.subcore_barrier()

    def step(j, carry):
        pltpu.sync_copy(onev, degsh.at[dstv.at[j]], add=True)
        return carry

    lax.fori_loop(0, NCH, step, 0)

    plsc.subcore_barrier()
    _writeback(cid, sid, degsh, dg0, dg1)


def _build_agg():
    return pl.kernel(
        _agg_body,
        out_type=[jax.ShapeDtypeStruct((N, DF), jnp.float32),
                  jax.ShapeDtypeStruct((N, DF), jnp.float32)],
        mesh=_sc_mesh(),
        scratch_types=[
            pltpu.VMEM((NCH, CHUNK), jnp.int32),
            pltpu.VMEM((NCH, CHUNK), jnp.int32),
            pltpu.VMEM((CHUNK, DF), jnp.float32),
            pltpu.VMEM_SHARED((NPAD, DF), jnp.float32),
            pltpu.SemaphoreType.DMA,
        ],
    )


def _build_deg():
    return pl.kernel(
        _deg_body,
        out_type=[jax.ShapeDtypeStruct((N, DW), jnp.float32),
                  jax.ShapeDtypeStruct((N, DW), jnp.float32)],
        mesh=_sc_mesh(),
        scratch_types=[
            pltpu.VMEM((NCH, CHUNK), jnp.int32),
            pltpu.VMEM((CHUNK, DW), jnp.float32),
            pltpu.VMEM_SHARED((NPAD, DW), jnp.float32),
        ],
    )


_sc_cache = {}


def _get_sc(name):
    if name not in _sc_cache:
        _sc_cache[name] = _build_agg() if name == "agg" else _build_deg()
    return _sc_cache[name]


def _mm(a, b):
    return lax.dot_general(
        a, b, (((1,), (0,)), ((), ())),
        preferred_element_type=jnp.float32,
        precision=lax.Precision.HIGHEST,
    )


def _mmT(a, b):
    return lax.dot_general(
        a, b, (((0,), (0,)), ((), ())),
        preferred_element_type=jnp.float32,
        precision=lax.Precision.HIGHEST,
    )


def _mmt(a, b):
    return lax.dot_general(
        a, b, (((1,), (1,)), ((), ())),
        preferred_element_type=jnp.float32,
        precision=lax.Precision.HIGHEST,
    )


def _leaky(v):
    return jnp.where(v >= 0, v, 0.01 * v)


def _tc1_body(p0, p1, d0, d1, x, wl, bl, wr, out):
    deg = jnp.maximum(jnp.max(d0[...] + d1[...], axis=1, keepdims=True), 1.0)
    mean = (p0[...] + p1[...]) / deg
    v = _mmt(mean, wl[...]) + bl[...] + _mmt(x[...], wr[...])
    out[...] = _leaky(v)


def _tc2_body(q0, q1, d0, d1, h1, bt, wl, bl, wr, k1ft, cwv, gmat, hgrp,
              t10, rsum, wm1, bm1, wm2, bm2, out, acc):
    i = pl.program_id(0)

    @pl.when(i == 0)
    def _zero():
        acc[...] = jnp.zeros_like(acc)

    deg = jnp.maximum(jnp.max(d0[...] + d1[...], axis=1, keepdims=True), 1.0)
    mean = (q0[...] + q1[...]) / deg
    h = _leaky(_mmt(mean, wl[...]) + bl[...] + _mmt(h1[...], wr[...]))

    kp = k1ft[...]
    cross = _mmt(h, kp)
    x2 = jnp.sum(h * h, axis=1, keepdims=True)
    k2n = jnp.sum(kp * kp, axis=1)[None, :]
    dist = jnp.maximum(x2 + k2n - 2.0 * cross, 0.0)
    t = 1.0 / (1.0 + dist)
    den = _mm(t, gmat[...])
    spre = (t / den) * cwv[...]
    s = _mm(spre, hgrp[...])
    m = jnp.max(s, axis=1, keepdims=True)
    e = jnp.exp(s - m)
    p = e / jnp.sum(e, axis=1, keepdims=True)

    oh = (bt[...] == lax.broadcasted_iota(jnp.int32, (BN, NGR), 1))
    oh = oh.astype(jnp.float32)
    q = _mm(oh, rsum[...]) * _mm(p, t10[...])
    acc[...] += _mmT(q, h)

    @pl.when(i == NB - 1)
    def _final():
        x1 = _leaky(_mmt(acc[...], wm1[...]) + bm1[...])
        g = _mm(rsum[...], x1)
        out[...] = _mmt(g, wm2[...]) + bm2[...]


def _bspec(shape):
    return pl.BlockSpec(shape, lambda i: (i, 0))


def _wspec(shape):
    return pl.BlockSpec(shape, lambda i: (0, 0))


_tc1 = pl.pallas_call(
    _tc1_body,
    grid=(NB,),
    in_specs=[
        _bspec((BN, DF)), _bspec((BN, DF)),
        _bspec((BN, DW)), _bspec((BN, DW)),
        _bspec((BN, DF)),
        _wspec((DF, DF)), _wspec((1, DF)), _wspec((DF, DF)),
    ],
    out_specs=_bspec((BN, DF)),
    out_shape=jax.ShapeDtypeStruct((N, DF), jnp.float32),
)

_tc2 = pl.pallas_call(
    _tc2_body,
    grid=(NB,),
    in_specs=[
        _bspec((BN, DF)), _bspec((BN, DF)),
        _bspec((BN, DW)), _bspec((BN, DW)),
        _bspec((BN, DF)), _bspec((BN, 1)),
        _wspec((DF, DF)), _wspec((1, DF)), _wspec((DF, DF)),
        _wspec((64, DF)), _wspec((1, 64)), _wspec((64, 64)), _wspec((64, 10)),
        _wspec((10, 160)), _wspec((16, 160)),
        _wspec((DF, DF)), _wspec((1, DF)), _wspec((10, DF)), _wspec((1, 10)),
    ],
    out_specs=pl.BlockSpec((NGR, 10), lambda i: (0, 0)),
    out_shape=jax.ShapeDtypeStruct((NGR, 10), jnp.float32),
    scratch_shapes=[pltpu.VMEM((NGR * 10, DF), jnp.float32)],
)

_EYE10 = np.eye(10, dtype=np.float32)
_G64 = np.zeros((64, 64), np.float32)
_G64[:50, :50] = np.kron(np.eye(5, dtype=np.float32), np.ones((10, 10), np.float32))
_G64[50:, 50:] = np.eye(14, dtype=np.float32)
_HGRP64 = np.zeros((64, 10), np.float32)
_HGRP64[:50] = np.tile(_EYE10, (5, 1))
_T10 = np.tile(_EYE10, (1, 16))
_RSUM = np.kron(np.eye(16, dtype=np.float32), np.ones((1, 10), np.float32))


def kernel(x, edge_index, batch, Wl1, bl1, Wr1, Wl2, bl2, Wr2,
           k1, cw1, Wm1, bm1, k2, cw2, Wm2, bm2):
    src3 = edge_index[0].reshape(NW, NCH, CHUNK)
    dst3 = edge_index[1].reshape(NW, NCH, CHUNK)
    znd = jnp.zeros((NPAD, DF), jnp.float32)
    zdg = jnp.zeros((NPAD, DW), jnp.float32)
    onesc = jnp.concatenate(
        [jnp.ones((CHUNK, 1), jnp.float32), jnp.zeros((CHUNK, DW - 1), jnp.float32)],
        axis=1)

    d0, d1 = _get_sc("deg")(dst3, zdg, onesc)
    p0, p1 = _get_sc("agg")(x, src3, dst3, znd)

    h1 = _tc1(p0, p1, d0, d1, x, Wl1, bl1[None, :], Wr1)

    q0, q1 = _get_sc("agg")(h1, src3, dst3, znd)

    k1f = k1.reshape(50, DF)
    k1p = jnp.concatenate([k1f, jnp.zeros((14, DF), jnp.float32)], axis=0)
    cwv = jnp.concatenate([jnp.repeat(cw1, 10), jnp.zeros((14,), jnp.float32)])[None, :]
    bt = batch.reshape(N, 1)

    out = _tc2(
        q0, q1, d0, d1, h1, bt,
        Wl2, bl2[None, :], Wr2,
        k1p, cwv, jnp.asarray(_G64), jnp.asarray(_HGRP64),
        jnp.asarray(_T10), jnp.asarray(_RSUM),
        Wm1, bm1[None, :], Wm2, bm2[None, :],
    )
    return out


# speedup vs baseline: 1.0845x; 1.0845x over previous
import numpy as np
import jax
import jax.numpy as jnp
from jax import lax
from jax.experimental import pallas as pl
from jax.experimental.pallas import tpu as pltpu
from jax.experimental.pallas import tpu_sc as plsc

N = 10000
E = 320000
DF = 128
NGR = 16
NC = 2
NS = 16
NW = NC * NS
EPW = E // NW
CHUNK = 125
NCH = EPW // CHUNK
NPAD = 10240
RPS = NPAD // NS
DW = 16
BN = 2000
NB = N // BN


def _sc_mesh():
    return plsc.VectorSubcoreMesh(
        core_axis_name="c", subcore_axis_name="s", num_cores=NC, num_subcores=NS)


def _agg_body(feat, src3, dst3, znd, out0, out1, srcv, dstv, buf, accsh, sem):
    cid = lax.axis_index("c")
    sid = lax.axis_index("s")
    wid = cid * NS + sid

    pltpu.sync_copy(src3.at[wid], srcv)
    pltpu.sync_copy(dst3.at[wid], dstv)

    r0 = sid * RPS
    pltpu.sync_copy(znd.at[pl.ds(r0, RPS)], accsh.at[pl.ds(r0, RPS)])
    plsc.subcore_barrier()

    def step(j, carry):
        pltpu.async_copy(feat.at[srcv.at[j]], buf, sem).wait()
        pltpu.sync_copy(buf, accsh.at[dstv.at[j]], add=True)
        return carry

    lax.fori_loop(0, NCH, step, 0)

    plsc.subcore_barrier()
    _writeback(cid, sid, accsh, out0, out1)


def _writeback(cid, sid, sh, out0, out1):
    r0 = sid * RPS
    nrows = N - 15 * RPS

    @pl.when(cid == 0)
    def _c0():
        @pl.when(sid < NS - 1)
        def _full():
            pltpu.sync_copy(sh.at[pl.ds(r0, RPS)], out0.at[pl.ds(r0, RPS)])

        @pl.when(sid == NS - 1)
        def _last():
            pltpu.sync_copy(sh.at[pl.ds(r0, nrows)], out0.at[pl.ds(r0, nrows)])

    @pl.when(cid == 1)
    def _c1():
        @pl.when(sid < NS - 1)
        def _full():
            pltpu.sync_copy(sh.at[pl.ds(r0, RPS)], out1.at[pl.ds(r0, RPS)])

        @pl.when(sid == NS - 1)
        def _last():
            pltpu.sync_copy(sh.at[pl.ds(r0, nrows)], out1.at[pl.ds(r0, nrows)])


def _deg_body(dst3, zdg, onesc, dg0, dg1, dstv, onev, degsh):
    cid = lax.axis_index("c")
    sid = lax.axis_index("s")
    wid = cid * NS + sid

    pltpu.sync_copy(dst3.at[wid], dstv)
    r0 = sid * RPS
    pltpu.sync_copy(zdg.at[pl.ds(r0, RPS)], degsh.at[pl.ds(r0, RPS)])
    pltpu.sync_copy(onesc, onev)
    plsc.subcore_barrier()

    def step(j, carry):
        pltpu.sync_copy(onev, degsh.at[dstv.at[j]], add=True)
        return carry

    lax.fori_loop(0, NCH, step, 0)

    plsc.subcore_barrier()
    _writeback(cid, sid, degsh, dg0, dg1)


def _build_agg():
    return pl.kernel(
        _agg_body,
        out_type=[jax.ShapeDtypeStruct((N, DF), jnp.float32),
                  jax.ShapeDtypeStruct((N, DF), jnp.float32)],
        mesh=_sc_mesh(),
        scratch_types=[
            pltpu.VMEM((NCH, CHUNK), jnp.int32),
            pltpu.VMEM((NCH, CHUNK), jnp.int32),
            pltpu.VMEM((CHUNK, DF), jnp.float32),
            pltpu.VMEM_SHARED((NPAD, DF), jnp.float32),
            pltpu.SemaphoreType.DMA,
        ],
    )


def _build_deg():
    return pl.kernel(
        _deg_body,
        out_type=[jax.ShapeDtypeStruct((N, DW), jnp.float32),
                  jax.ShapeDtypeStruct((N, DW), jnp.float32)],
        mesh=_sc_mesh(),
        scratch_types=[
            pltpu.VMEM((NCH, CHUNK), jnp.int32),
            pltpu.VMEM((CHUNK, DW), jnp.float32),
            pltpu.VMEM_SHARED((NPAD, DW), jnp.float32),
        ],
    )


_sc_cache = {}


def _get_sc(name):
    if name not in _sc_cache:
        _sc_cache[name] = _build_agg() if name == "agg" else _build_deg()
    return _sc_cache[name]


def _mm(a, b):
    return lax.dot_general(
        a, b, (((1,), (0,)), ((), ())),
        preferred_element_type=jnp.float32,
        precision=lax.Precision.HIGHEST,
    )


def _mmT(a, b):
    return lax.dot_general(
        a, b, (((0,), (0,)), ((), ())),
        preferred_element_type=jnp.float32,
        precision=lax.Precision.HIGHEST,
    )


def _mmt(a, b):
    return lax.dot_general(
        a, b, (((1,), (1,)), ((), ())),
        preferred_element_type=jnp.float32,
        precision=lax.Precision.HIGHEST,
    )


def _leaky(v):
    return jnp.where(v >= 0, v, 0.01 * v)


def _tc1_body(p0, p1, d0, d1, x, wl, bl, wr, out):
    deg = jnp.maximum(jnp.max(d0[...] + d1[...], axis=1, keepdims=True), 1.0)
    mean = (p0[...] + p1[...]) / deg
    v = _mmt(mean, wl[...]) + bl[...] + _mmt(x[...], wr[...])
    out[...] = _leaky(v)


def _tc2_body(q0, q1, d0, d1, h1, bt, wl, bl, wr, k1ft, cwv, gmat, hgrp,
              t10, rsum, wm1, bm1, wm2, bm2, out, acc):
    i = pl.program_id(0)

    @pl.when(i == 0)
    def _zero():
        acc[...] = jnp.zeros_like(acc)

    deg = jnp.maximum(jnp.max(d0[...] + d1[...], axis=1, keepdims=True), 1.0)
    mean = (q0[...] + q1[...]) / deg
    h = _leaky(_mmt(mean, wl[...]) + bl[...] + _mmt(h1[...], wr[...]))

    kp = k1ft[...]
    cross = _mmt(h, kp)
    x2 = jnp.sum(h * h, axis=1, keepdims=True)
    k2n = jnp.sum(kp * kp, axis=1)[None, :]
    dist = jnp.maximum(x2 + k2n - 2.0 * cross, 0.0)
    t = 1.0 / (1.0 + dist)
    den = _mm(t, gmat[...])
    spre = (t / den) * cwv[...]
    s = _mm(spre, hgrp[...])
    m = jnp.max(s, axis=1, keepdims=True)
    e = jnp.exp(s - m)
    p = e / jnp.sum(e, axis=1, keepdims=True)

    oh = (bt[...] == lax.broadcasted_iota(jnp.int32, (BN, NGR), 1))
    oh = oh.astype(jnp.float32)
    q = _mm(oh, rsum[...]) * _mm(p, t10[...])
    acc[...] += _mmT(q, h)

    @pl.when(i == NB - 1)
    def _final():
        x1 = _leaky(_mmt(acc[...], wm1[...]) + bm1[...])
        g = _mm(rsum[...], x1)
        out[...] = _mmt(g, wm2[...]) + bm2[...]


def _bspec(shape):
    return pl.BlockSpec(shape, lambda i: (i, 0))


def _wspec(shape):
    return pl.BlockSpec(shape, lambda i: (0, 0))


_tc1 = pl.pallas_call(
    _tc1_body,
    grid=(NB,),
    in_specs=[
        _bspec((BN, DF)), _bspec((BN, DF)),
        _bspec((BN, DW)), _bspec((BN, DW)),
        _bspec((BN, DF)),
        _wspec((DF, DF)), _wspec((1, DF)), _wspec((DF, DF)),
    ],
    out_specs=_bspec((BN, DF)),
    out_shape=jax.ShapeDtypeStruct((N, DF), jnp.float32),
)

_tc2 = pl.pallas_call(
    _tc2_body,
    grid=(NB,),
    in_specs=[
        _bspec((BN, DF)), _bspec((BN, DF)),
        _bspec((BN, DW)), _bspec((BN, DW)),
        _bspec((BN, DF)), _bspec((BN, 1)),
        _wspec((DF, DF)), _wspec((1, DF)), _wspec((DF, DF)),
        _wspec((64, DF)), _wspec((1, 64)), _wspec((64, 64)), _wspec((64, 10)),
        _wspec((10, 160)), _wspec((16, 160)),
        _wspec((DF, DF)), _wspec((1, DF)), _wspec((10, DF)), _wspec((1, 10)),
    ],
    out_specs=pl.BlockSpec((NGR, 10), lambda i: (0, 0)),
    out_shape=jax.ShapeDtypeStruct((NGR, 10), jnp.float32),
    scratch_shapes=[pltpu.VMEM((NGR * 10, DF), jnp.float32)],
)

_EYE10 = np.eye(10, dtype=np.float32)
_G64 = np.zeros((64, 64), np.float32)
_G64[:50, :50] = np.kron(np.eye(5, dtype=np.float32), np.ones((10, 10), np.float32))
_G64[50:, 50:] = np.eye(14, dtype=np.float32)
_HGRP64 = np.zeros((64, 10), np.float32)
_HGRP64[:50] = np.tile(_EYE10, (5, 1))
_T10 = np.tile(_EYE10, (1, 16))
_RSUM = np.kron(np.eye(16, dtype=np.float32), np.ones((1, 10), np.float32))


def kernel(x, edge_index, batch, Wl1, bl1, Wr1, Wl2, bl2, Wr2,
           k1, cw1, Wm1, bm1, k2, cw2, Wm2, bm2):
    src3 = edge_index[0].reshape(NW, NCH, CHUNK)
    dst3 = edge_index[1].reshape(NW, NCH, CHUNK)
    znd = jnp.zeros((NPAD, DF), jnp.float32)
    zdg = jnp.zeros((NPAD, DW), jnp.float32)
    onesc = jnp.concatenate(
        [jnp.ones((CHUNK, 1), jnp.float32), jnp.zeros((CHUNK, DW - 1), jnp.float32)],
        axis=1)

    d0, d1 = _get_sc("deg")(dst3, zdg, onesc)
    p0, p1 = _get_sc("agg")(x, src3, dst3, znd)

    h1 = _tc1(p0, p1, d0, d1, x, Wl1, bl1[None, :], Wr1)

    q0, q1 = _get_sc("agg")(h1, src3, dst3, znd)

    k1f = k1.reshape(50, DF)
    k1p = jnp.concatenate([k1f, jnp.zeros((14, DF), jnp.float32)], axis=0)
    cwv = jnp.concatenate([jnp.repeat(cw1, 10), jnp.zeros((14,), jnp.float32)])[None, :]
    bt = batch.reshape(N, 1)

    out = _tc2(
        q0, q1, d0, d1, h1, bt,
        Wl2, bl2[None, :], Wr2,
        k1p, cwv, jnp.asarray(_G64), jnp.asarray(_HGRP64),
        jnp.asarray(_T10), jnp.asarray(_RSUM),
        Wm1, bm1[None, :], Wm2, bm2[None, :],
    )
    return out
